# baseline (speedup 1.0000x reference)
import functools

import jax
import jax.numpy as jnp
from jax import lax
from jax.experimental import pallas as pl
from jax.experimental.pallas import tpu as pltpu
from jax.experimental.pallas import tpu_sc as plsc

_EPS = 1e-5
_NC = 2
_NS = 16
_NW = _NC * _NS


def _mesh():
    return plsc.VectorSubcoreMesh(
        core_axis_name="c", subcore_axis_name="s",
        num_cores=_NC, num_subcores=_NS)


_SC_PARAMS = pltpu.CompilerParams(needs_layout_passes=False)


def _sc_degree(row, n_pad):
    e = row.shape[0]
    epw = e // _NW
    k = 80
    nch = epw // k
    rpt = n_pad // _NS
    zr = 128
    d = 128

    @functools.partial(
        pl.kernel,
        out_type=jax.ShapeDtypeStruct((_NC * n_pad, d), jnp.float32),
        mesh=_mesh(),
        scratch_types=[
            pltpu.VMEM((zr, d), jnp.float32),
            pltpu.VMEM((k, d), jnp.float32),
            pltpu.VMEM((k,), jnp.int32),
            pltpu.VMEM((k,), jnp.int32),
            pltpu.VMEM_SHARED((n_pad, d), jnp.float32),
            pltpu.SemaphoreType.DMA,
            pltpu.SemaphoreType.DMA,
            pltpu.SemaphoreType.DMA,
            pltpu.SemaphoreType.DMA,
        ],
    )
    def deg_kernel(row_hbm, deg_hbm, zbuf, ones, idxv0, idxv1, acc,
                   si0, si1, ss0, ss1):
        cid = lax.axis_index("c")
        sid = lax.axis_index("s")
        wid = sid * _NC + cid
        idxv = (idxv0, idxv1)
        si = (si0, si1)
        ss = (ss0, ss1)

        @pl.loop(0, zr)
        def _(j):
            for f in range(d // 16):
                zbuf[j, pl.ds(f * 16, 16)] = jnp.zeros((16,), jnp.float32)

        @pl.loop(0, k)
        def _(j):
            for f in range(d // 16):
                ones[j, pl.ds(f * 16, 16)] = jnp.ones((16,), jnp.float32)

        for i in range(rpt // zr):
            pltpu.sync_copy(zbuf, acc.at[pl.ds(sid * rpt + i * zr, zr)])
        plsc.subcore_barrier()

        def load(ci, b):
            base = wid * epw + ci * k
            pltpu.async_copy(row_hbm.at[pl.ds(base, k)], idxv[b], si[b])

        def scat(b):
            pltpu.make_async_copy(row_hbm.at[pl.ds(0, k)], idxv[b],
                                  si[b]).wait()
            pltpu.async_copy(ones, acc.at[idxv[b]], ss[b], add=True)

        def drain(b):
            pltpu.make_async_copy(ones, acc.at[idxv[b]], ss[b]).wait()

        load(0, 0)
        load(1, 1)
        scat(0)

        @pl.loop(0, (nch - 2) // 2)
        def _(p):
            ci = p * 2
            for off in range(2):
                scat(1 - off)
                drain(off)
                load(ci + off + 2, off)

        if (nch - 2) % 2:
            scat((nch - 2) % 2)
            drain((nch - 3) % 2)
            load(nch - 1, (nch - 3) % 2)
            scat((nch - 1) % 2)
            drain((nch - 2) % 2)
            drain((nch - 1) % 2)
        else:
            scat((nch - 1) % 2)
            drain((nch - 2) % 2)
            drain((nch - 1) % 2)

        plsc.subcore_barrier()
        for i in range(rpt // zr):
            r0 = sid * rpt + i * zr
            pltpu.sync_copy(acc.at[pl.ds(r0, zr)],
                            deg_hbm.at[pl.ds(cid * n_pad + r0, zr)])

    return deg_kernel(row)


def _sc_disr(row, dis):
    e = row.shape[0]
    n_nodes = dis.shape[0]
    epw = e // _NW

    @functools.partial(
        pl.kernel,
        out_type=jax.ShapeDtypeStruct((e,), jnp.float32),
        mesh=_mesh(),
        compiler_params=_SC_PARAMS,
        scratch_types=[
            pltpu.VMEM((n_nodes,), jnp.float32),
            pltpu.VMEM((epw,), jnp.int32),
            pltpu.VMEM((epw,), jnp.float32),
        ],
    )
    def disr_kernel(row_hbm, dis_hbm, disr_hbm, disv, rv, nb):
        cid = lax.axis_index("c")
        sid = lax.axis_index("s")
        wid = sid * _NC + cid
        base = wid * epw
        pltpu.sync_copy(dis_hbm, disv)
        pltpu.sync_copy(row_hbm.at[pl.ds(base, epw)], rv)

        @pl.loop(0, epw // 16)
        def _(j):
            r16 = rv[pl.ds(j * 16, 16)]
            nb[pl.ds(j * 16, 16)] = plsc.load_gather(disv, [r16])

        pltpu.sync_copy(nb, disr_hbm.at[pl.ds(base, epw)])

    return disr_kernel(row, dis)


def _sc_conv(hs, es, row, col, n_pad):
    n_nodes, d = hs.shape
    e = row.shape[0]
    epw = e // _NW
    k = 80
    nch = epw // k
    rpt = n_pad // _NS
    zr = 32
    nf = d // 16

    @functools.partial(
        pl.kernel,
        out_type=jax.ShapeDtypeStruct((_NC * n_pad, d), jnp.float32),
        mesh=_mesh(),
        scratch_types=[
            pltpu.VMEM((k,), jnp.int32),
            pltpu.VMEM((k,), jnp.int32),
            pltpu.VMEM((k,), jnp.int32),
            pltpu.VMEM((k,), jnp.int32),
            pltpu.VMEM((k, d), jnp.float32),
            pltpu.VMEM((k, d), jnp.float32),
            pltpu.VMEM((k, d), jnp.float32),
            pltpu.VMEM((k, d), jnp.float32),
            pltpu.VMEM((zr, d), jnp.float32),
            pltpu.VMEM_SHARED((n_pad, d), jnp.float32),
            pltpu.SemaphoreType.DMA,
            pltpu.SemaphoreType.DMA,
            pltpu.SemaphoreType.DMA,
            pltpu.SemaphoreType.DMA,
        ],
    )
    def conv_kernel(hs_hbm, es_hbm, row_hbm, col_hbm, out_hbm,
                    rowv0, rowv1, colv0, colv1, g0, g1, eb0, eb1,
                    zbuf, acc, sg0, sg1, se0, se1):
        cid = lax.axis_index("c")
        sid = lax.axis_index("s")
        wid = sid * _NC + cid
        rowv = (rowv0, rowv1)
        colv = (colv0, colv1)
        g = (g0, g1)
        eb = (eb0, eb1)
        sg = (sg0, sg1)
        se = (se0, se1)

        @pl.loop(0, zr)
        def _(j):
            for f in range(nf):
                zbuf[j, pl.ds(f * 16, 16)] = jnp.zeros((16,), jnp.float32)

        for i in range(rpt // zr):
            pltpu.sync_copy(zbuf, acc.at[pl.ds(sid * rpt + i * zr, zr)])
        plsc.subcore_barrier()

        def issue(ci, b):
            base = wid * epw + ci * k
            pltpu.sync_copy(row_hbm.at[pl.ds(base, k)], rowv[b])
            pltpu.async_copy(hs_hbm.at[rowv[b]], g[b], sg[b])
            pltpu.async_copy(es_hbm.at[pl.ds(base, k)], eb[b], se[b])
            pltpu.sync_copy(col_hbm.at[pl.ds(base, k)], colv[b])

        def finish(b):
            pltpu.make_async_copy(hs_hbm.at[rowv[b]], g[b], sg[b]).wait()
            pltpu.make_async_copy(es_hbm.at[pl.ds(0, k)], eb[b],
                                  se[b]).wait()

            @pl.loop(0, k)
            def _(j):
                for f in range(nf):
                    sl = pl.ds(f * 16, 16)
                    g[b][j, sl] = jnp.maximum(
                        g[b][j, sl] + eb[b][j, sl], 0.0)

            pltpu.sync_copy(g[b], acc.at[colv[b]], add=True)

        issue(0, 0)

        @pl.loop(0, nch - 1, step=2)
        def _(si):
            for b in range(2):
                issue(si + b + 1, 1 - b)
                finish(b)

        finish((nch - 1) % 2)

        plsc.subcore_barrier()
        for i in range(rpt // zr):
            r0 = sid * rpt + i * zr
            pltpu.sync_copy(acc.at[pl.ds(r0, zr)],
                            out_hbm.at[pl.ds(cid * n_pad + r0, zr)])

    return conv_kernel(hs, es, row, col)


def _dot(a, b):
    return jnp.dot(a, b, preferred_element_type=jnp.float32,
                   precision=lax.Precision.HIGHEST)


def _tc_hdis(x, w, b, degacc, n_pad):
    n, _ = x.shape
    hid = w.shape[1]

    def body(x_ref, w_ref, b_ref, d_ref, hs_ref, dis_ref):
        deg = d_ref[0:n, 0:1] + d_ref[n_pad:n_pad + n, 0:1]
        dis = lax.rsqrt(deg + 1.0)
        dis_ref[...] = dis
        hs_ref[...] = (_dot(x_ref[...], w_ref[...]) + b_ref[...]) * dis

    return pl.pallas_call(
        body,
        out_shape=(jax.ShapeDtypeStruct((n, hid), jnp.float32),
                   jax.ShapeDtypeStruct((n, 1), jnp.float32)),
    )(x, w, b.reshape(1, hid), degacc)


def _tc_edge3(edge_attr, we3, be3, disr2d):
    e, dedge = edge_attr.shape
    hid = we3.shape[1] // 3
    blk = 4000
    grid = e // blk

    def body(a_ref, w_ref, b_ref, s_ref, o1_ref, o2_ref, o3_ref):
        m = (_dot(a_ref[...], w_ref[...]) + b_ref[...]) * s_ref[...]
        o1_ref[...] = m[:, 0:hid]
        o2_ref[...] = m[:, hid:2 * hid]
        o3_ref[...] = m[:, 2 * hid:3 * hid]

    return pl.pallas_call(
        body,
        grid=(grid,),
        in_specs=[
            pl.BlockSpec((blk, dedge), lambda i: (i, 0)),
            pl.BlockSpec((dedge, 3 * hid), lambda i: (0, 0)),
            pl.BlockSpec((1, 3 * hid), lambda i: (0, 0)),
            pl.BlockSpec((blk, 1), lambda i: (i, 0)),
        ],
        out_specs=[pl.BlockSpec((blk, hid), lambda i: (i, 0))] * 3,
        out_shape=[jax.ShapeDtypeStruct((e, hid), jnp.float32)] * 3,
    )(edge_attr, we3, be3, disr2d)


def _tc_mid(part, dis2d, w, b, n, scale_out):
    n_pad = part.shape[0] // 2
    hid = w.shape[1]

    def body(p_ref, dis_ref, w_ref, b_ref, o_ref):
        dis = dis_ref[...]
        s = (p_ref[0:n, :] + p_ref[n_pad:n_pad + n, :]) * dis
        r = jnp.maximum(s, 0.0)
        m = jnp.mean(r, axis=0, keepdims=True)
        cvec = r - m
        v = jnp.mean(cvec * cvec, axis=0, keepdims=True)
        hb = cvec * lax.rsqrt(v + _EPS)
        o = _dot(hb, w_ref[...]) + b_ref[...]
        if scale_out:
            o = o * dis
        o_ref[...] = o

    return pl.pallas_call(
        body,
        out_shape=jax.ShapeDtypeStruct((n, hid), jnp.float32),
    )(part, dis2d, w, b.reshape(1, hid))


def kernel(x, edge_index, edge_attr, W1, b1, We1, be1, W2, b2, We2, be2,
           W3, b3, We3, be3, Wl, bl):
    n = x.shape[0]
    n_pad = -(-n // 1280) * 1280
    row = edge_index[0]
    col = edge_index[1]
    e = row.shape[0]

    degacc = _sc_degree(row, n_pad)
    hs, dis2d = _tc_hdis(x, W1, b1, degacc, n_pad)
    dis = dis2d.reshape(n)
    disr2d = _sc_disr(row, dis).reshape(e, 1)

    we3 = jnp.concatenate([We1, We2, We3], axis=1)
    be3 = jnp.concatenate([be1, be2, be3]).reshape(1, -1)
    es1, es2, es3 = _tc_edge3(edge_attr, we3, be3, disr2d)

    part = _sc_conv(hs, es1, row, col, n_pad)
    hs = _tc_mid(part, dis2d, W2, b2, n, True)
    part = _sc_conv(hs, es2, row, col, n_pad)
    hs = _tc_mid(part, dis2d, W3, b3, n, True)
    part = _sc_conv(hs, es3, row, col, n_pad)
    return _tc_mid(part, dis2d, Wl, bl, n, False)


# speedup vs baseline: 1.0612x; 1.0612x over previous
import functools

import jax
import jax.numpy as jnp
from jax import lax
from jax.experimental import pallas as pl
from jax.experimental.pallas import tpu as pltpu
from jax.experimental.pallas import tpu_sc as plsc

_EPS = 1e-5
_NC = 2
_NS = 16
_NW = _NC * _NS


def _mesh():
    return plsc.VectorSubcoreMesh(
        core_axis_name="c", subcore_axis_name="s",
        num_cores=_NC, num_subcores=_NS)


_SC_PARAMS = pltpu.CompilerParams(needs_layout_passes=False)


def _sc_degree(row, n_pad):
    e = row.shape[0]
    epw = e // _NW
    k = 80
    nch = epw // k
    rpt = n_pad // _NS
    zr = 128
    d = 128

    @functools.partial(
        pl.kernel,
        out_type=jax.ShapeDtypeStruct((_NC * n_pad, d), jnp.float32),
        mesh=_mesh(),
        scratch_types=[
            pltpu.VMEM((zr, d), jnp.float32),
            pltpu.VMEM((k, d), jnp.float32),
            pltpu.VMEM((k,), jnp.int32),
            pltpu.VMEM((k,), jnp.int32),
            pltpu.VMEM_SHARED((n_pad, d), jnp.float32),
            pltpu.SemaphoreType.DMA,
            pltpu.SemaphoreType.DMA,
            pltpu.SemaphoreType.DMA,
            pltpu.SemaphoreType.DMA,
        ],
    )
    def deg_kernel(row_hbm, deg_hbm, zbuf, ones, idxv0, idxv1, acc,
                   si0, si1, ss0, ss1):
        cid = lax.axis_index("c")
        sid = lax.axis_index("s")
        wid = sid * _NC + cid
        idxv = (idxv0, idxv1)
        si = (si0, si1)
        ss = (ss0, ss1)

        @pl.loop(0, zr)
        def _(j):
            for f in range(d // 16):
                zbuf[j, pl.ds(f * 16, 16)] = jnp.zeros((16,), jnp.float32)

        @pl.loop(0, k)
        def _(j):
            for f in range(d // 16):
                ones[j, pl.ds(f * 16, 16)] = jnp.ones((16,), jnp.float32)

        for i in range(rpt // zr):
            pltpu.sync_copy(zbuf, acc.at[pl.ds(sid * rpt + i * zr, zr)])
        plsc.subcore_barrier()

        def load(ci, b):
            base = wid * epw + ci * k
            pltpu.async_copy(row_hbm.at[pl.ds(base, k)], idxv[b], si[b])

        def scat(b):
            pltpu.make_async_copy(row_hbm.at[pl.ds(0, k)], idxv[b],
                                  si[b]).wait()
            pltpu.async_copy(ones, acc.at[idxv[b]], ss[b], add=True)

        def drain(b):
            pltpu.make_async_copy(ones, acc.at[idxv[b]], ss[b]).wait()

        load(0, 0)
        load(1, 1)
        scat(0)

        @pl.loop(0, (nch - 2) // 2)
        def _(p):
            ci = p * 2
            for off in range(2):
                scat(1 - off)
                drain(off)
                load(ci + off + 2, off)

        if (nch - 2) % 2:
            scat((nch - 2) % 2)
            drain((nch - 3) % 2)
            load(nch - 1, (nch - 3) % 2)
            scat((nch - 1) % 2)
            drain((nch - 2) % 2)
            drain((nch - 1) % 2)
        else:
            scat((nch - 1) % 2)
            drain((nch - 2) % 2)
            drain((nch - 1) % 2)

        plsc.subcore_barrier()
        for i in range(rpt // zr):
            r0 = sid * rpt + i * zr
            pltpu.sync_copy(acc.at[pl.ds(r0, zr)],
                            deg_hbm.at[pl.ds(cid * n_pad + r0, zr)])

    return deg_kernel(row)


def _sc_disr(row, dis):
    e = row.shape[0]
    n_nodes = dis.shape[0]
    epw = e // _NW

    @functools.partial(
        pl.kernel,
        out_type=jax.ShapeDtypeStruct((e,), jnp.float32),
        mesh=_mesh(),
        compiler_params=_SC_PARAMS,
        scratch_types=[
            pltpu.VMEM((n_nodes,), jnp.float32),
            pltpu.VMEM((epw,), jnp.int32),
            pltpu.VMEM((epw,), jnp.float32),
        ],
    )
    def disr_kernel(row_hbm, dis_hbm, disr_hbm, disv, rv, nb):
        cid = lax.axis_index("c")
        sid = lax.axis_index("s")
        wid = sid * _NC + cid
        base = wid * epw
        pltpu.sync_copy(dis_hbm, disv)
        pltpu.sync_copy(row_hbm.at[pl.ds(base, epw)], rv)

        @pl.loop(0, epw // 16)
        def _(j):
            r16 = rv[pl.ds(j * 16, 16)]
            nb[pl.ds(j * 16, 16)] = plsc.load_gather(disv, [r16])

        pltpu.sync_copy(nb, disr_hbm.at[pl.ds(base, epw)])

    return disr_kernel(row, dis)


def _sc_conv(hs, es, row, col, n_pad):
    n_nodes, d = hs.shape
    e = row.shape[0]
    epw = e // _NW
    k = 80
    nch = epw // k
    rpt = n_pad // _NS
    zr = 32
    nf = d // 16

    @functools.partial(
        pl.kernel,
        out_type=jax.ShapeDtypeStruct((_NC * n_pad, d), jnp.float32),
        mesh=_mesh(),
        scratch_types=[
            pltpu.VMEM((k,), jnp.int32),
            pltpu.VMEM((k,), jnp.int32),
            pltpu.VMEM((k,), jnp.int32),
            pltpu.VMEM((k,), jnp.int32),
            pltpu.VMEM((k, d), jnp.float32),
            pltpu.VMEM((k, d), jnp.float32),
            pltpu.VMEM((k, d), jnp.float32),
            pltpu.VMEM((k, d), jnp.float32),
            pltpu.VMEM((zr, d), jnp.float32),
            pltpu.VMEM_SHARED((n_pad, d), jnp.float32),
            pltpu.SemaphoreType.DMA,
            pltpu.SemaphoreType.DMA,
            pltpu.SemaphoreType.DMA,
            pltpu.SemaphoreType.DMA,
        ],
    )
    def conv_kernel(hs_hbm, es_hbm, row_hbm, col_hbm, out_hbm,
                    rowv0, rowv1, colv0, colv1, g0, g1, eb0, eb1,
                    zbuf, acc, sg0, sg1, se0, se1):
        cid = lax.axis_index("c")
        sid = lax.axis_index("s")
        wid = sid * _NC + cid
        rowv = (rowv0, rowv1)
        colv = (colv0, colv1)
        g = (g0, g1)
        eb = (eb0, eb1)
        sg = (sg0, sg1)
        se = (se0, se1)

        @pl.loop(0, zr)
        def _(j):
            for f in range(nf):
                zbuf[j, pl.ds(f * 16, 16)] = jnp.zeros((16,), jnp.float32)

        for i in range(rpt // zr):
            pltpu.sync_copy(zbuf, acc.at[pl.ds(sid * rpt + i * zr, zr)])
        plsc.subcore_barrier()

        def issue(ci, b):
            base = wid * epw + ci * k
            pltpu.sync_copy(row_hbm.at[pl.ds(base, k)], rowv[b])
            pltpu.async_copy(hs_hbm.at[rowv[b]], g[b], sg[b])
            pltpu.async_copy(es_hbm.at[pl.ds(base, k)], eb[b], se[b])
            pltpu.sync_copy(col_hbm.at[pl.ds(base, k)], colv[b])

        def finish(b):
            pltpu.make_async_copy(hs_hbm.at[rowv[b]], g[b], sg[b]).wait()
            pltpu.make_async_copy(es_hbm.at[pl.ds(0, k)], eb[b],
                                  se[b]).wait()

            @pl.loop(0, k)
            def _(j):
                for f in range(nf):
                    sl = pl.ds(f * 16, 16)
                    g[b][j, sl] = jnp.maximum(
                        g[b][j, sl] + eb[b][j, sl], 0.0)

            pltpu.sync_copy(g[b], acc.at[colv[b]], add=True)

        issue(0, 0)

        @pl.loop(0, nch - 1, step=2)
        def _(si):
            for b in range(2):
                issue(si + b + 1, 1 - b)
                finish(b)

        finish((nch - 1) % 2)

        plsc.subcore_barrier()
        for i in range(rpt // zr):
            r0 = sid * rpt + i * zr
            pltpu.sync_copy(acc.at[pl.ds(r0, zr)],
                            out_hbm.at[pl.ds(cid * n_pad + r0, zr)])

    return conv_kernel(hs, es, row, col)


def _dot(a, b):
    return jnp.dot(a, b, preferred_element_type=jnp.float32,
                   precision=lax.Precision.HIGHEST)


def _tc_h(x, w, b):
    n, _ = x.shape
    hid = w.shape[1]

    def body(x_ref, w_ref, b_ref, h_ref):
        h_ref[...] = _dot(x_ref[...], w_ref[...]) + b_ref[...]

    return pl.pallas_call(
        body,
        out_shape=jax.ShapeDtypeStruct((n, hid), jnp.float32),
    )(x, w, b.reshape(1, hid))


def _tc_dis(h, degacc, n_pad):
    n, hid = h.shape

    def body(h_ref, d_ref, hs_ref, dis_ref):
        deg = d_ref[0:n, 0:1] + d_ref[n_pad:n_pad + n, 0:1]
        dis = lax.rsqrt(deg + 1.0)
        dis_ref[...] = dis
        hs_ref[...] = h_ref[...] * dis

    return pl.pallas_call(
        body,
        out_shape=(jax.ShapeDtypeStruct((n, hid), jnp.float32),
                   jax.ShapeDtypeStruct((n, 1), jnp.float32)),
    )(h, degacc)


def _tc_edge(edge_attr, we, be, disr2d):
    e, dedge = edge_attr.shape
    hid = we.shape[1]
    blk = 4000
    grid = e // blk

    def body(a_ref, w_ref, b_ref, s_ref, o_ref):
        o_ref[...] = (_dot(a_ref[...], w_ref[...]) + b_ref[...]) * s_ref[...]

    return pl.pallas_call(
        body,
        grid=(grid,),
        in_specs=[
            pl.BlockSpec((blk, dedge), lambda i: (i, 0)),
            pl.BlockSpec((dedge, hid), lambda i: (0, 0)),
            pl.BlockSpec((1, hid), lambda i: (0, 0)),
            pl.BlockSpec((blk, 1), lambda i: (i, 0)),
        ],
        out_specs=pl.BlockSpec((blk, hid), lambda i: (i, 0)),
        out_shape=jax.ShapeDtypeStruct((e, hid), jnp.float32),
    )(edge_attr, we, be.reshape(1, hid), disr2d)


def _tc_mid(part, dis2d, w, b, n, scale_out):
    n_pad = part.shape[0] // 2
    hid = w.shape[1]

    def body(p_ref, dis_ref, w_ref, b_ref, o_ref):
        dis = dis_ref[...]
        s = (p_ref[0:n, :] + p_ref[n_pad:n_pad + n, :]) * dis
        r = jnp.maximum(s, 0.0)
        m = jnp.mean(r, axis=0, keepdims=True)
        cvec = r - m
        v = jnp.mean(cvec * cvec, axis=0, keepdims=True)
        hb = cvec * lax.rsqrt(v + _EPS)
        o = _dot(hb, w_ref[...]) + b_ref[...]
        if scale_out:
            o = o * dis
        o_ref[...] = o

    return pl.pallas_call(
        body,
        out_shape=jax.ShapeDtypeStruct((n, hid), jnp.float32),
    )(part, dis2d, w, b.reshape(1, hid))


def kernel(x, edge_index, edge_attr, W1, b1, We1, be1, W2, b2, We2, be2,
           W3, b3, We3, be3, Wl, bl):
    n = x.shape[0]
    n_pad = -(-n // 1280) * 1280
    row = edge_index[0]
    col = edge_index[1]
    e = row.shape[0]

    h1 = _tc_h(x, W1, b1)
    degacc = _sc_degree(row, n_pad)
    hs, dis2d = _tc_dis(h1, degacc, n_pad)
    dis = dis2d.reshape(n)
    disr2d = _sc_disr(row, dis).reshape(e, 1)

    part = _sc_conv(hs, _tc_edge(edge_attr, We1, be1, disr2d),
                    row, col, n_pad)
    hs = _tc_mid(part, dis2d, W2, b2, n, True)
    part = _sc_conv(hs, _tc_edge(edge_attr, We2, be2, disr2d),
                    row, col, n_pad)
    hs = _tc_mid(part, dis2d, W3, b3, n, True)
    part = _sc_conv(hs, _tc_edge(edge_attr, We3, be3, disr2d),
                    row, col, n_pad)
    return _tc_mid(part, dis2d, Wl, bl, n, False)


# speedup vs baseline: 1.1923x; 1.1236x over previous
import functools

import jax
import jax.numpy as jnp
from jax import lax
from jax.experimental import pallas as pl
from jax.experimental.pallas import tpu as pltpu
from jax.experimental.pallas import tpu_sc as plsc

_EPS = 1e-5
_NC = 2
_NS = 16
_NW = _NC * _NS


def _mesh():
    return plsc.VectorSubcoreMesh(
        core_axis_name="c", subcore_axis_name="s",
        num_cores=_NC, num_subcores=_NS)


_SC_PARAMS = pltpu.CompilerParams(needs_layout_passes=False)


def _sc_degree(row, n_pad):
    e = row.shape[0]
    epw = e // _NW
    k = 80
    nch = epw // k
    rpt = n_pad // _NS
    zr = 128
    d = 128

    @functools.partial(
        pl.kernel,
        out_type=jax.ShapeDtypeStruct((_NC * n_pad, d), jnp.float32),
        mesh=_mesh(),
        scratch_types=[
            pltpu.VMEM((zr, d), jnp.float32),
            pltpu.VMEM((k, d), jnp.float32),
            pltpu.VMEM((k,), jnp.int32),
            pltpu.VMEM((k,), jnp.int32),
            pltpu.VMEM_SHARED((n_pad, d), jnp.float32),
            pltpu.SemaphoreType.DMA,
            pltpu.SemaphoreType.DMA,
            pltpu.SemaphoreType.DMA,
            pltpu.SemaphoreType.DMA,
        ],
    )
    def deg_kernel(row_hbm, deg_hbm, zbuf, ones, idxv0, idxv1, acc,
                   si0, si1, ss0, ss1):
        cid = lax.axis_index("c")
        sid = lax.axis_index("s")
        wid = sid * _NC + cid
        idxv = (idxv0, idxv1)
        si = (si0, si1)
        ss = (ss0, ss1)

        @pl.loop(0, zr)
        def _(j):
            for f in range(d // 16):
                zbuf[j, pl.ds(f * 16, 16)] = jnp.zeros((16,), jnp.float32)

        @pl.loop(0, k)
        def _(j):
            for f in range(d // 16):
                ones[j, pl.ds(f * 16, 16)] = jnp.ones((16,), jnp.float32)

        for i in range(rpt // zr):
            pltpu.sync_copy(zbuf, acc.at[pl.ds(sid * rpt + i * zr, zr)])
        plsc.subcore_barrier()

        def load(ci, b):
            base = wid * epw + ci * k
            pltpu.async_copy(row_hbm.at[pl.ds(base, k)], idxv[b], si[b])

        def scat(b):
            pltpu.make_async_copy(row_hbm.at[pl.ds(0, k)], idxv[b],
                                  si[b]).wait()
            pltpu.async_copy(ones, acc.at[idxv[b]], ss[b], add=True)

        def drain(b):
            pltpu.make_async_copy(ones, acc.at[idxv[b]], ss[b]).wait()

        load(0, 0)
        load(1, 1)
        scat(0)

        @pl.loop(0, (nch - 2) // 2)
        def _(p):
            ci = p * 2
            for off in range(2):
                scat(1 - off)
                drain(off)
                load(ci + off + 2, off)

        if (nch - 2) % 2:
            scat((nch - 2) % 2)
            drain((nch - 3) % 2)
            load(nch - 1, (nch - 3) % 2)
            scat((nch - 1) % 2)
            drain((nch - 2) % 2)
            drain((nch - 1) % 2)
        else:
            scat((nch - 1) % 2)
            drain((nch - 2) % 2)
            drain((nch - 1) % 2)

        plsc.subcore_barrier()
        for i in range(rpt // zr):
            r0 = sid * rpt + i * zr
            pltpu.sync_copy(acc.at[pl.ds(r0, zr)],
                            deg_hbm.at[pl.ds(cid * n_pad + r0, zr)])

    return deg_kernel(row)


def _sc_disr(row, dis):
    e = row.shape[0]
    n_nodes = dis.shape[0]
    epw = e // _NW

    @functools.partial(
        pl.kernel,
        out_type=jax.ShapeDtypeStruct((e,), jnp.float32),
        mesh=_mesh(),
        compiler_params=_SC_PARAMS,
        scratch_types=[
            pltpu.VMEM((n_nodes,), jnp.float32),
            pltpu.VMEM((epw,), jnp.int32),
            pltpu.VMEM((epw,), jnp.float32),
        ],
    )
    def disr_kernel(row_hbm, dis_hbm, disr_hbm, disv, rv, nb):
        cid = lax.axis_index("c")
        sid = lax.axis_index("s")
        wid = sid * _NC + cid
        base = wid * epw
        pltpu.sync_copy(dis_hbm, disv)
        pltpu.sync_copy(row_hbm.at[pl.ds(base, epw)], rv)

        @pl.loop(0, epw // 16)
        def _(j):
            r16 = rv[pl.ds(j * 16, 16)]
            nb[pl.ds(j * 16, 16)] = plsc.load_gather(disv, [r16])

        pltpu.sync_copy(nb, disr_hbm.at[pl.ds(base, epw)])

    return disr_kernel(row, dis)


def _sc_conv(hs, es, row, col, n_pad):
    n_nodes, d = hs.shape
    e = row.shape[0]
    epw = e // _NW
    k = 80
    nch = epw // k
    rpt = n_pad // _NS
    zr = 32
    nf = d // 16

    scratch = (
        [pltpu.VMEM((k,), jnp.int32)] * 4 +
        [pltpu.VMEM((k,), jnp.int32)] * 4 +
        [pltpu.VMEM((k, d), jnp.float32)] * 4 +
        [
            pltpu.VMEM((zr, d), jnp.float32),
            pltpu.VMEM_SHARED((n_pad, d), jnp.float32),
        ] +
        [pltpu.SemaphoreType.DMA] * 12
    )

    @functools.partial(
        pl.kernel,
        out_type=jax.ShapeDtypeStruct((_NC * n_pad, d), jnp.float32),
        mesh=_mesh(),
        scratch_types=scratch,
    )
    def conv_kernel(hs_hbm, es_hbm, row_hbm, col_hbm, out_hbm,
                    rowv0, rowv1, rowv2, rowv3, colv0, colv1, colv2, colv3,
                    g0, g1, eb0, eb1, zbuf, acc, sg0, sg1, se0, se1,
                    sir0, sir1, sir2, sir3, sic0, sic1, sic2, sic3):
        cid = lax.axis_index("c")
        sid = lax.axis_index("s")
        wid = sid * _NC + cid
        rowv = (rowv0, rowv1, rowv2, rowv3)
        colv = (colv0, colv1, colv2, colv3)
        g = (g0, g1)
        eb = (eb0, eb1)
        sg = (sg0, sg1)
        se = (se0, se1)
        sir = (sir0, sir1, sir2, sir3)
        sic = (sic0, sic1, sic2, sic3)

        @pl.loop(0, zr)
        def _(j):
            for f in range(nf):
                zbuf[j, pl.ds(f * 16, 16)] = jnp.zeros((16,), jnp.float32)

        for i in range(rpt // zr):
            pltpu.sync_copy(zbuf, acc.at[pl.ds(sid * rpt + i * zr, zr)])
        plsc.subcore_barrier()

        def pre(ci, b):
            base = wid * epw + ci * k
            pltpu.async_copy(row_hbm.at[pl.ds(base, k)], rowv[b], sir[b])
            pltpu.async_copy(col_hbm.at[pl.ds(base, k)], colv[b], sic[b])

        def issue(ci, b):
            base = wid * epw + ci * k
            pltpu.make_async_copy(row_hbm.at[pl.ds(0, k)], rowv[b],
                                  sir[b]).wait()
            pltpu.async_copy(hs_hbm.at[rowv[b]], g[b], sg[b])
            pltpu.async_copy(es_hbm.at[pl.ds(base, k)], eb[b], se[b])

        def finish(b):
            pltpu.make_async_copy(hs_hbm.at[rowv[b]], g[b], sg[b]).wait()
            pltpu.make_async_copy(es_hbm.at[pl.ds(0, k)], eb[b],
                                  se[b]).wait()

            @pl.loop(0, k)
            def _(j):
                for f in range(nf):
                    sl = pl.ds(f * 16, 16)
                    g[b][j, sl] = jnp.maximum(
                        g[b][j, sl] + eb[b][j, sl], 0.0)

            pltpu.make_async_copy(col_hbm.at[pl.ds(0, k)], colv[b],
                                  sic[b]).wait()
            pltpu.sync_copy(g[b], acc.at[colv[b]], add=True)

        pre(0, 0)
        pre(1, 1)
        issue(0, 0)

        @pl.loop(0, (nch - 2) // 2)
        def _(p):
            ci = p * 2
            for off in range(2):
                issue(ci + off + 1, 1 - off)
                finish(off)
                pre(ci + off + 2, off)

        if (nch - 2) % 2:
            c0 = nch - 3
            issue(c0 + 1, (c0 + 1) % 2)
            finish(c0 % 2)
            pre(c0 + 2, c0 % 2)
        issue(nch - 1, (nch - 1) % 2)
        finish((nch - 2) % 2)
        finish((nch - 1) % 2)

        plsc.subcore_barrier()
        for i in range(rpt // zr):
            r0 = sid * rpt + i * zr
            pltpu.sync_copy(acc.at[pl.ds(r0, zr)],
                            out_hbm.at[pl.ds(cid * n_pad + r0, zr)])

    return conv_kernel(hs, es, row, col)


def _dot(a, b):
    return jnp.dot(a, b, preferred_element_type=jnp.float32,
                   precision=lax.Precision.HIGHEST)


def _tc_h(x, w, b):
    n, _ = x.shape
    hid = w.shape[1]

    def body(x_ref, w_ref, b_ref, h_ref):
        h_ref[...] = _dot(x_ref[...], w_ref[...]) + b_ref[...]

    return pl.pallas_call(
        body,
        out_shape=jax.ShapeDtypeStruct((n, hid), jnp.float32),
    )(x, w, b.reshape(1, hid))


def _tc_dis(h, degacc, n_pad):
    n, hid = h.shape

    def body(h_ref, d_ref, hs_ref, dis_ref):
        deg = d_ref[0:n, 0:1] + d_ref[n_pad:n_pad + n, 0:1]
        dis = lax.rsqrt(deg + 1.0)
        dis_ref[...] = dis
        hs_ref[...] = h_ref[...] * dis

    return pl.pallas_call(
        body,
        out_shape=(jax.ShapeDtypeStruct((n, hid), jnp.float32),
                   jax.ShapeDtypeStruct((n, 1), jnp.float32)),
    )(h, degacc)


def _tc_edge(edge_attr, we, be, disr2d):
    e, dedge = edge_attr.shape
    hid = we.shape[1]
    blk = 4000
    grid = e // blk

    def body(a_ref, w_ref, b_ref, s_ref, o_ref):
        o_ref[...] = (_dot(a_ref[...], w_ref[...]) + b_ref[...]) * s_ref[...]

    return pl.pallas_call(
        body,
        grid=(grid,),
        in_specs=[
            pl.BlockSpec((blk, dedge), lambda i: (i, 0)),
            pl.BlockSpec((dedge, hid), lambda i: (0, 0)),
            pl.BlockSpec((1, hid), lambda i: (0, 0)),
            pl.BlockSpec((blk, 1), lambda i: (i, 0)),
        ],
        out_specs=pl.BlockSpec((blk, hid), lambda i: (i, 0)),
        out_shape=jax.ShapeDtypeStruct((e, hid), jnp.float32),
    )(edge_attr, we, be.reshape(1, hid), disr2d)


def _tc_mid(part, dis2d, w, b, n, scale_out):
    n_pad = part.shape[0] // 2
    hid = w.shape[1]

    def body(p_ref, dis_ref, w_ref, b_ref, o_ref):
        dis = dis_ref[...]
        s = (p_ref[0:n, :] + p_ref[n_pad:n_pad + n, :]) * dis
        r = jnp.maximum(s, 0.0)
        m = jnp.mean(r, axis=0, keepdims=True)
        cvec = r - m
        v = jnp.mean(cvec * cvec, axis=0, keepdims=True)
        hb = cvec * lax.rsqrt(v + _EPS)
        o = _dot(hb, w_ref[...]) + b_ref[...]
        if scale_out:
            o = o * dis
        o_ref[...] = o

    return pl.pallas_call(
        body,
        out_shape=jax.ShapeDtypeStruct((n, hid), jnp.float32),
    )(part, dis2d, w, b.reshape(1, hid))


def kernel(x, edge_index, edge_attr, W1, b1, We1, be1, W2, b2, We2, be2,
           W3, b3, We3, be3, Wl, bl):
    n = x.shape[0]
    n_pad = -(-n // 1280) * 1280
    row = edge_index[0]
    col = edge_index[1]
    e = row.shape[0]

    h1 = _tc_h(x, W1, b1)
    degacc = _sc_degree(row, n_pad)
    hs, dis2d = _tc_dis(h1, degacc, n_pad)
    dis = dis2d.reshape(n)
    disr2d = _sc_disr(row, dis).reshape(e, 1)

    part = _sc_conv(hs, _tc_edge(edge_attr, We1, be1, disr2d),
                    row, col, n_pad)
    hs = _tc_mid(part, dis2d, W2, b2, n, True)
    part = _sc_conv(hs, _tc_edge(edge_attr, We2, be2, disr2d),
                    row, col, n_pad)
    hs = _tc_mid(part, dis2d, W3, b3, n, True)
    part = _sc_conv(hs, _tc_edge(edge_attr, We3, be3, disr2d),
                    row, col, n_pad)
    return _tc_mid(part, dis2d, Wl, bl, n, False)
